# Initial kernel scaffold; baseline (speedup 1.0000x reference)
import jax
import jax.numpy as jnp
from jax.experimental import pallas as pl


def kernel(features, batch_idx, W):
    raise NotImplementedError("write your pallas kernel here")


# speedup vs baseline: 8.8197x; 8.8197x over previous
import functools

import jax
import jax.numpy as jnp
from jax.experimental import pallas as pl
from jax.experimental.pallas import tpu as pltpu

B = 16


def _p1(bidx_ref, feat_ref, sums_ref, cnt_ref):
    i = pl.program_id(0)
    rb = feat_ref.shape[0]
    b = bidx_ref[0, 0, :]
    onehot = (b[:, None] == jax.lax.broadcasted_iota(jnp.int32, (rb, B), 1)
              ).astype(jnp.float32)
    part = jax.lax.dot_general(onehot, feat_ref[...],
                               (((0,), (0,)), ((), ())),
                               preferred_element_type=jnp.float32)
    pcnt = jnp.sum(onehot, axis=0)[:, None]

    @pl.when(i == 0)
    def _():
        sums_ref[...] = jnp.zeros_like(sums_ref)
        cnt_ref[...] = jnp.zeros_like(cnt_ref)

    sums_ref[...] += part
    cnt_ref[...] += jnp.broadcast_to(pcnt, cnt_ref.shape)


def _p2(sums_ref, cnt_ref, w_ref, gates_ref):
    m = sums_ref[...] / jnp.maximum(cnt_ref[...], 1.0)
    w0 = w_ref[0, 0]
    w1 = w_ref[0, 1]
    w2 = w_ref[0, 2]
    zero = jnp.zeros((m.shape[0], 1), jnp.float32)
    left = jnp.concatenate([zero, m[:, :-1]], axis=1)
    right = jnp.concatenate([m[:, 1:], zero], axis=1)
    y = w0 * left + w1 * m + w2 * right
    gates_ref[...] = jax.nn.sigmoid(y)


def _p3(bidx_ref, feat_ref, gates_ref, out_ref):
    rb = feat_ref.shape[0]
    b = bidx_ref[0, 0, :]
    onehot = (b[:, None] == jax.lax.broadcasted_iota(jnp.int32, (rb, B), 1)
              ).astype(jnp.float32)
    g = jax.lax.dot_general(onehot, gates_ref[...],
                            (((1,), (0,)), ((), ())),
                            preferred_element_type=jnp.float32)
    out_ref[...] = feat_ref[...] * g


def _row_block(n):
    for rb in range(min(n, 4096), 7, -1):
        if n % rb == 0 and rb % 8 == 0:
            return rb
    return n


@functools.partial(jax.jit, static_argnames=())
def kernel(features, batch_idx, W):
    n, c = features.shape
    rb = _row_block(n)
    nb = n // rb
    bidx3 = batch_idx.reshape(nb, 1, rb)

    sums, cnt = pl.pallas_call(
        _p1,
        grid=(nb,),
        in_specs=[
            pl.BlockSpec((1, 1, rb), lambda i: (i, 0, 0)),
            pl.BlockSpec((rb, c), lambda i: (i, 0)),
        ],
        out_specs=[
            pl.BlockSpec((B, c), lambda i: (0, 0)),
            pl.BlockSpec((B, c), lambda i: (0, 0)),
        ],
        out_shape=[
            jax.ShapeDtypeStruct((B, c), jnp.float32),
            jax.ShapeDtypeStruct((B, c), jnp.float32),
        ],
    )(bidx3, features)

    gates = pl.pallas_call(
        _p2,
        out_shape=jax.ShapeDtypeStruct((B, c), jnp.float32),
    )(sums, cnt, W.reshape(1, 3))

    out = pl.pallas_call(
        _p3,
        grid=(nb,),
        in_specs=[
            pl.BlockSpec((1, 1, rb), lambda i: (i, 0, 0)),
            pl.BlockSpec((rb, c), lambda i: (i, 0)),
            pl.BlockSpec((B, c), lambda i: (0, 0)),
        ],
        out_specs=pl.BlockSpec((rb, c), lambda i: (i, 0)),
        out_shape=jax.ShapeDtypeStruct((n, c), jnp.float32),
    )(bidx3, features, gates)
    return out
